# baseline (speedup 1.0000x reference)
import functools

import jax
import jax.numpy as jnp
from jax.experimental import pallas as pl

NUM_MATCHES = 20
ALPHA = 0.75
INV_COEFF = 25.0
VAR_COEFF = 15.0
COV_COEFF = 1.0
GAMMA = 1.0

B, P, D_LOC, D_GLOB = 32, 576, 384, 2048
BIG_I32 = 1 << 30


def _min_body(x1m_ref, x2m_ref, x1l_ref, x2l_ref,
              frmin_ref, fcmin_ref, lrmin_ref, lcmin_ref,
              n1f_ref, n2f_ref, n1l_ref, n2l_ref):
    for (aref, bref, rmin, cmin, n1, n2) in (
            (x1m_ref, x2m_ref, frmin_ref, fcmin_ref, n1f_ref, n2f_ref),
            (x1l_ref, x2l_ref, lrmin_ref, lcmin_ref, n1l_ref, n2l_ref)):
        a = aref[0]
        b = bref[0]
        a2 = jnp.sum(a * a, axis=1, keepdims=True)
        b2row = jnp.transpose(jnp.sum(b * b, axis=1, keepdims=True))
        nab2 = jax.lax.dot_general(
            a.astype(jnp.bfloat16),
            b.astype(jnp.bfloat16) * jnp.bfloat16(-2.0),
            (((1,), (1,)), ((), ())),
            preferred_element_type=jnp.float32)
        d2 = (a2 + b2row) + nab2
        dist = jnp.sqrt(jnp.maximum(d2, 0.0))
        di = jax.lax.bitcast_convert_type(dist, jnp.int32)
        rmin[0, :, 0] = jax.lax.bitcast_convert_type(
            jnp.min(di, axis=1), jnp.float32)
        cmin[0, 0, :] = jax.lax.bitcast_convert_type(
            jnp.min(di, axis=0), jnp.float32)
        n1[0, :, 0] = a2[:, 0]
        n2[0, 0, :] = b2row[0]


def _topk_body(vals_ref, chin_ref, fi_ref, firsts_ref):
    s = 4 * B
    iota = jax.lax.broadcasted_iota(jnp.int32, (s, P), 1)
    chin = chin_ref[...]
    v = vals_ref[...]
    fis = []
    firsts = []
    for _ in range(NUM_MATCHES):
        m = jnp.min(v, axis=1, keepdims=True)
        first = jnp.min(jnp.where(v == m, iota, BIG_I32),
                        axis=1, keepdims=True)
        onehot = iota == first
        fi = jnp.sum(jnp.where(onehot, chin, 0.0), axis=1, keepdims=True)
        fis.append(fi)
        firsts.append(first)
        v = jnp.where(onehot, jnp.inf, v)
    fi_ref[...] = jnp.concatenate(fis, axis=1)
    firsts_ref[...] = jnp.concatenate(firsts, axis=1)


def _argmin_body(x1m_ref, x2m_ref, x1l_ref, x2l_ref,
                 n1f_ref, n2f_ref, n1l_ref, n2l_ref,
                 firsts_ref, ch1_ref, ch2_ref, fc_ref):
    x1mb = x1m_ref[0].astype(jnp.bfloat16)
    x2mb = x2m_ref[0].astype(jnp.bfloat16)
    x1lb = x1l_ref[0].astype(jnp.bfloat16)
    x2lb = x2l_ref[0].astype(jnp.bfloat16)
    fs = firsts_ref[0]
    iota = jax.lax.broadcasted_iota(jnp.int32, (NUM_MATCHES, P), 1)
    fcs = []
    for d, (inb, a2row, b2row, candb, chc) in enumerate((
            (x1mb, n1f_ref, n2f_ref, x2mb, ch2_ref),
            (x2mb, n2f_ref, n1f_ref, x1mb, ch1_ref),
            (x1lb, n1l_ref, n2l_ref, x2lb, ch2_ref),
            (x2lb, n2l_ref, n1l_ref, x1lb, ch1_ref))):
        first_d = fs[:, d:d + 1]
        oh = iota == first_d
        a2s = jnp.sum(
            jnp.where(oh, jnp.broadcast_to(a2row[0], (NUM_MATCHES, P)), 0.0),
            axis=1, keepdims=True)
        asel = jax.lax.dot_general(
            oh.astype(jnp.bfloat16), inb,
            (((1,), (0,)), ((), ())),
            preferred_element_type=jnp.float32)
        nab2 = jax.lax.dot_general(
            (asel * -2.0).astype(jnp.bfloat16), candb,
            (((1,), (1,)), ((), ())),
            preferred_element_type=jnp.float32)
        dist = jnp.sqrt(jnp.maximum((a2s + b2row[0]) + nab2, 0.0))
        m = jnp.min(dist, axis=1, keepdims=True)
        cidx = jnp.min(jnp.where(dist == m, iota, BIG_I32),
                       axis=1, keepdims=True)
        oh2 = iota == cidx
        fc = jnp.sum(
            jnp.where(oh2, jnp.broadcast_to(chc[0], (NUM_MATCHES, P)), 0.0),
            axis=1, keepdims=True)
        fcs.append(fc)
    fc_ref[0] = jnp.concatenate(fcs, axis=1)


def _global_half(x):
    xc = x - jnp.sum(x, axis=0, keepdims=True) / B
    xcc = xc - jnp.sum(xc, axis=0, keepdims=True) / B
    std = jnp.sqrt(jnp.sum(xcc * xcc, axis=0, keepdims=True) / (B - 1))
    var = jnp.sum(jnp.maximum(GAMMA - std, 0.0)) / D_GLOB
    xcb = xc.astype(jnp.bfloat16)
    xcb32 = xcb.astype(jnp.float32)
    colss = jnp.sum(xcb32 * xcb32, axis=0, keepdims=True)
    g = jax.lax.dot_general(
        xcb, xcb, (((1,), (1,)), ((), ())),
        preferred_element_type=jnp.float32)
    s_all = jnp.sum(g * g)
    s_diag = jnp.sum(colss * colss)
    cov = (s_all - s_diag) / ((B - 1.0) * (B - 1.0)) / D_GLOB
    return var, cov


def _stats_body(fi_ref, fc_ref, g1_ref, g2_ref,
                loss_ref, gl_ref, loc_ref, feat_ref):
    fi_all = fi_ref[...]
    fc_all = fc_ref[...]
    terms = []
    for d in range(4):
        fi = fi_all[d * B:(d + 1) * B]
        fc = fc_all[d * B:(d + 1) * B]
        inv = INV_COEFF * jnp.sum((fi - fc) ** 2) / (B * NUM_MATCHES)
        mu_i = jnp.sum(fi, axis=0, keepdims=True) / B
        std_i = jnp.sqrt(jnp.sum((fi - mu_i) ** 2, axis=0,
                                 keepdims=True) / (B - 1))
        mu_c = jnp.sum(fc, axis=0, keepdims=True) / B
        std_c = jnp.sqrt(jnp.sum((fc - mu_c) ** 2, axis=0,
                                 keepdims=True) / (B - 1))
        var = VAR_COEFF * (jnp.sum(jnp.maximum(GAMMA - std_i, 0.0))
                           + jnp.sum(jnp.maximum(GAMMA - std_c, 0.0))
                           ) / (2.0 * NUM_MATCHES)
        terms.append(inv + var)
    f12, f21, l12, l21 = terms
    feat = (f12 + f21) / 2.0
    loc = (l12 + l21) / 2.0

    g1 = g1_ref[...]
    g2 = g2_ref[...]
    inv_g = INV_COEFF * jnp.sum((g1 - g2) ** 2) / (B * D_GLOB)
    var1, cov1 = _global_half(g1)
    var2, cov2 = _global_half(g2)
    global_loss = (inv_g + VAR_COEFF * (var1 + var2) / 2.0
                   + COV_COEFF * (cov1 + cov2) / 2.0)

    loss = ALPHA * global_loss + (1.0 - ALPHA) * (feat + loc) / 2.0
    loss_ref[...] = jnp.reshape(loss, (1, 1))
    gl_ref[...] = jnp.reshape(global_loss, (1, 1))
    loc_ref[...] = jnp.reshape(loc, (1, 1))
    feat_ref[...] = jnp.reshape(feat, (1, 1))


@functools.partial(jax.jit, static_argnames=("interpret",))
def _run(x1_maps, x2_maps, x1_glob, x2_glob, x1_locations, x2_locations,
         interpret=False):
    rowspec = pl.BlockSpec((1, 1, P), lambda b: (b, 0, 0))
    colspec = pl.BlockSpec((1, P, 1), lambda b: (b, 0, 0))
    maps_spec = pl.BlockSpec((1, P, D_LOC), lambda b: (b, 0, 0))
    loc_spec = pl.BlockSpec((1, P, 2), lambda b: (b, 0, 0))
    rowshape = jax.ShapeDtypeStruct((B, 1, P), jnp.float32)
    colshape = jax.ShapeDtypeStruct((B, P, 1), jnp.float32)

    mins = pl.pallas_call(
        _min_body,
        grid=(B,),
        in_specs=[maps_spec, maps_spec, loc_spec, loc_spec],
        out_specs=[colspec, rowspec, colspec, rowspec,
                   colspec, rowspec, colspec, rowspec],
        out_shape=[colshape, rowshape, colshape, rowshape,
                   colshape, rowshape, colshape, rowshape],
        interpret=interpret,
    )(x1_maps, x2_maps, x1_locations, x2_locations)
    frmin, fcmin, lrmin, lcmin, n1f, n2f, n1l, n2l = mins
    n1f = n1f.reshape(B, 1, P)
    n1l = n1l.reshape(B, 1, P)

    vals = jnp.concatenate(
        [a.reshape(B, P) for a in (frmin, fcmin, lrmin, lcmin)], axis=0)
    ch1 = x1_maps[:, :, 0]
    ch2 = x2_maps[:, :, 0]
    chin = jnp.concatenate([ch1, ch2, ch1, ch2], axis=0)

    fi_all, firsts = pl.pallas_call(
        _topk_body,
        out_shape=[jax.ShapeDtypeStruct((4 * B, NUM_MATCHES), jnp.float32),
                   jax.ShapeDtypeStruct((4 * B, NUM_MATCHES), jnp.int32)],
        interpret=interpret,
    )(vals, chin)

    firsts_t = firsts.reshape(4, B, NUM_MATCHES).transpose(1, 2, 0)

    fc3 = pl.pallas_call(
        _argmin_body,
        grid=(B,),
        in_specs=[maps_spec, maps_spec, loc_spec, loc_spec,
                  rowspec, rowspec, rowspec, rowspec,
                  pl.BlockSpec((1, NUM_MATCHES, 4), lambda b: (b, 0, 0)),
                  rowspec, rowspec],
        out_specs=pl.BlockSpec((1, NUM_MATCHES, 4), lambda b: (b, 0, 0)),
        out_shape=jax.ShapeDtypeStruct((B, NUM_MATCHES, 4), jnp.float32),
        interpret=interpret,
    )(x1_maps, x2_maps, x1_locations, x2_locations,
      n1f, n2f, n1l, n2l, firsts_t,
      ch1.reshape(B, 1, P), ch2.reshape(B, 1, P))

    fc_all = fc3.transpose(2, 0, 1).reshape(4 * B, NUM_MATCHES)

    out = pl.pallas_call(
        _stats_body,
        out_shape=[jax.ShapeDtypeStruct((1, 1), jnp.float32)] * 4,
        interpret=interpret,
    )(fi_all, fc_all, x1_glob, x2_glob)
    loss, gl, loc, feat = (o[0, 0] for o in out)
    return loss, gl, loc, feat


def kernel(x1_maps, x2_maps, x1_glob, x2_glob, x1_locations, x2_locations):
    return _run(x1_maps, x2_maps, x1_glob, x2_glob,
                x1_locations, x2_locations)


# speedup vs baseline: 1.0482x; 1.0482x over previous
import functools

import jax
import jax.numpy as jnp
from jax.experimental import pallas as pl

NUM_MATCHES = 20
ALPHA = 0.75
INV_COEFF = 25.0
VAR_COEFF = 15.0
COV_COEFF = 1.0
GAMMA = 1.0

B, P, D_LOC, D_GLOB = 32, 576, 384, 2048
BIG_I32 = 1 << 30


def _min_body(x1m_ref, x2m_ref, x1l_ref, x2l_ref,
              frmin_ref, fcmin_ref, lrmin_ref, lcmin_ref,
              n1f_ref, n2f_ref, n1l_ref, n2l_ref):
    for (aref, bref, rmin, cmin, n1, n2) in (
            (x1m_ref, x2m_ref, frmin_ref, fcmin_ref, n1f_ref, n2f_ref),
            (x1l_ref, x2l_ref, lrmin_ref, lcmin_ref, n1l_ref, n2l_ref)):
        a = aref[0]
        b = bref[0]
        a2 = jnp.sum(a * a, axis=1, keepdims=True)
        b2row = jnp.transpose(jnp.sum(b * b, axis=1, keepdims=True))
        nab2 = jax.lax.dot_general(
            a.astype(jnp.bfloat16),
            b.astype(jnp.bfloat16) * jnp.bfloat16(-2.0),
            (((1,), (1,)), ((), ())),
            preferred_element_type=jnp.float32)
        d2 = (a2 + b2row) + nab2
        dist = jnp.sqrt(jnp.maximum(d2, 0.0))
        di = jax.lax.bitcast_convert_type(dist, jnp.int32)
        rmin[0, 0, :] = jax.lax.bitcast_convert_type(
            jnp.min(di, axis=1), jnp.float32)
        cmin[0, 0, :] = jax.lax.bitcast_convert_type(
            jnp.min(di, axis=0), jnp.float32)
        n1[0, 0, :] = jnp.transpose(a2)[0]
        n2[0, 0, :] = b2row[0]


def _topk_body(vals_ref, chin_ref, fi_ref, firsts_ref):
    s = 4 * B
    iota = jax.lax.broadcasted_iota(jnp.int32, (s, P), 1)
    chin = chin_ref[...]
    v = vals_ref[...]
    fis = []
    firsts = []
    for _ in range(NUM_MATCHES):
        m = jnp.min(v, axis=1, keepdims=True)
        first = jnp.min(jnp.where(v == m, iota, BIG_I32),
                        axis=1, keepdims=True)
        onehot = iota == first
        fi = jnp.sum(jnp.where(onehot, chin, 0.0), axis=1, keepdims=True)
        fis.append(fi)
        firsts.append(first)
        v = jnp.where(onehot, jnp.inf, v)
    fi_ref[...] = jnp.concatenate(fis, axis=1)
    firsts_ref[...] = jnp.concatenate(firsts, axis=1)


def _argmin_body(x1m_ref, x2m_ref, x1l_ref, x2l_ref,
                 n1f_ref, n2f_ref, n1l_ref, n2l_ref,
                 firsts_ref, ch1_ref, ch2_ref, fc_ref):
    x1mb = x1m_ref[0].astype(jnp.bfloat16)
    x2mb = x2m_ref[0].astype(jnp.bfloat16)
    x1lb = x1l_ref[0].astype(jnp.bfloat16)
    x2lb = x2l_ref[0].astype(jnp.bfloat16)
    fs = firsts_ref[0]
    iota = jax.lax.broadcasted_iota(jnp.int32, (NUM_MATCHES, P), 1)
    fcs = []
    for d, (inb, a2row, b2row, candb, chc) in enumerate((
            (x1mb, n1f_ref, n2f_ref, x2mb, ch2_ref),
            (x2mb, n2f_ref, n1f_ref, x1mb, ch1_ref),
            (x1lb, n1l_ref, n2l_ref, x2lb, ch2_ref),
            (x2lb, n2l_ref, n1l_ref, x1lb, ch1_ref))):
        first_d = fs[:, d:d + 1]
        oh = iota == first_d
        a2s = jnp.sum(
            jnp.where(oh, jnp.broadcast_to(a2row[0], (NUM_MATCHES, P)), 0.0),
            axis=1, keepdims=True)
        asel = jax.lax.dot_general(
            oh.astype(jnp.bfloat16), inb,
            (((1,), (0,)), ((), ())),
            preferred_element_type=jnp.float32)
        nab2 = jax.lax.dot_general(
            (asel * -2.0).astype(jnp.bfloat16), candb,
            (((1,), (1,)), ((), ())),
            preferred_element_type=jnp.float32)
        dist = jnp.sqrt(jnp.maximum((a2s + b2row[0]) + nab2, 0.0))
        m = jnp.min(dist, axis=1, keepdims=True)
        cidx = jnp.min(jnp.where(dist == m, iota, BIG_I32),
                       axis=1, keepdims=True)
        oh2 = iota == cidx
        fc = jnp.sum(
            jnp.where(oh2, jnp.broadcast_to(chc[0], (NUM_MATCHES, P)), 0.0),
            axis=1, keepdims=True)
        fcs.append(fc)
    fc_ref[0] = jnp.concatenate(fcs, axis=1)


def _global_half(x):
    xc = x - jnp.sum(x, axis=0, keepdims=True) / B
    xcc = xc - jnp.sum(xc, axis=0, keepdims=True) / B
    std = jnp.sqrt(jnp.sum(xcc * xcc, axis=0, keepdims=True) / (B - 1))
    var = jnp.sum(jnp.maximum(GAMMA - std, 0.0)) / D_GLOB
    xcb = xc.astype(jnp.bfloat16)
    xcb32 = xcb.astype(jnp.float32)
    colss = jnp.sum(xcb32 * xcb32, axis=0, keepdims=True)
    g = jax.lax.dot_general(
        xcb, xcb, (((1,), (1,)), ((), ())),
        preferred_element_type=jnp.float32)
    s_all = jnp.sum(g * g)
    s_diag = jnp.sum(colss * colss)
    cov = (s_all - s_diag) / ((B - 1.0) * (B - 1.0)) / D_GLOB
    return var, cov


def _stats_body(fi_ref, fc_ref, g1_ref, g2_ref,
                loss_ref, gl_ref, loc_ref, feat_ref):
    fi_all = fi_ref[...]
    fc_all = fc_ref[...]
    terms = []
    for d in range(4):
        fi = fi_all[d * B:(d + 1) * B]
        fc = fc_all[d * B:(d + 1) * B]
        inv = INV_COEFF * jnp.sum((fi - fc) ** 2) / (B * NUM_MATCHES)
        mu_i = jnp.sum(fi, axis=0, keepdims=True) / B
        std_i = jnp.sqrt(jnp.sum((fi - mu_i) ** 2, axis=0,
                                 keepdims=True) / (B - 1))
        mu_c = jnp.sum(fc, axis=0, keepdims=True) / B
        std_c = jnp.sqrt(jnp.sum((fc - mu_c) ** 2, axis=0,
                                 keepdims=True) / (B - 1))
        var = VAR_COEFF * (jnp.sum(jnp.maximum(GAMMA - std_i, 0.0))
                           + jnp.sum(jnp.maximum(GAMMA - std_c, 0.0))
                           ) / (2.0 * NUM_MATCHES)
        terms.append(inv + var)
    f12, f21, l12, l21 = terms
    feat = (f12 + f21) / 2.0
    loc = (l12 + l21) / 2.0

    g1 = g1_ref[...]
    g2 = g2_ref[...]
    inv_g = INV_COEFF * jnp.sum((g1 - g2) ** 2) / (B * D_GLOB)
    var1, cov1 = _global_half(g1)
    var2, cov2 = _global_half(g2)
    global_loss = (inv_g + VAR_COEFF * (var1 + var2) / 2.0
                   + COV_COEFF * (cov1 + cov2) / 2.0)

    loss = ALPHA * global_loss + (1.0 - ALPHA) * (feat + loc) / 2.0
    loss_ref[...] = jnp.reshape(loss, (1, 1))
    gl_ref[...] = jnp.reshape(global_loss, (1, 1))
    loc_ref[...] = jnp.reshape(loc, (1, 1))
    feat_ref[...] = jnp.reshape(feat, (1, 1))


@functools.partial(jax.jit, static_argnames=("interpret",))
def _run(x1_maps, x2_maps, x1_glob, x2_glob, x1_locations, x2_locations,
         interpret=False):
    rowspec = pl.BlockSpec((1, 1, P), lambda b: (b, 0, 0))
    colspec = pl.BlockSpec((1, P, 1), lambda b: (b, 0, 0))
    maps_spec = pl.BlockSpec((1, P, D_LOC), lambda b: (b, 0, 0))
    loc_spec = pl.BlockSpec((1, P, 2), lambda b: (b, 0, 0))
    rowshape = jax.ShapeDtypeStruct((B, 1, P), jnp.float32)
    colshape = jax.ShapeDtypeStruct((B, P, 1), jnp.float32)

    mins = pl.pallas_call(
        _min_body,
        grid=(B,),
        in_specs=[maps_spec, maps_spec, loc_spec, loc_spec],
        out_specs=[rowspec] * 8,
        out_shape=[rowshape] * 8,
        interpret=interpret,
    )(x1_maps, x2_maps, x1_locations, x2_locations)
    frmin, fcmin, lrmin, lcmin, n1f, n2f, n1l, n2l = mins

    vals = jnp.concatenate(
        [a.reshape(B, P) for a in (frmin, fcmin, lrmin, lcmin)], axis=0)
    ch1 = x1_maps[:, :, 0]
    ch2 = x2_maps[:, :, 0]
    chin = jnp.concatenate([ch1, ch2, ch1, ch2], axis=0)

    fi_all, firsts = pl.pallas_call(
        _topk_body,
        out_shape=[jax.ShapeDtypeStruct((4 * B, NUM_MATCHES), jnp.float32),
                   jax.ShapeDtypeStruct((4 * B, NUM_MATCHES), jnp.int32)],
        interpret=interpret,
    )(vals, chin)

    firsts_t = firsts.reshape(4, B, NUM_MATCHES).transpose(1, 2, 0)

    fc3 = pl.pallas_call(
        _argmin_body,
        grid=(B,),
        in_specs=[maps_spec, maps_spec, loc_spec, loc_spec,
                  rowspec, rowspec, rowspec, rowspec,
                  pl.BlockSpec((1, NUM_MATCHES, 4), lambda b: (b, 0, 0)),
                  rowspec, rowspec],
        out_specs=pl.BlockSpec((1, NUM_MATCHES, 4), lambda b: (b, 0, 0)),
        out_shape=jax.ShapeDtypeStruct((B, NUM_MATCHES, 4), jnp.float32),
        interpret=interpret,
    )(x1_maps, x2_maps, x1_locations, x2_locations,
      n1f, n2f, n1l, n2l, firsts_t,
      ch1.reshape(B, 1, P), ch2.reshape(B, 1, P))

    fc_all = fc3.transpose(2, 0, 1).reshape(4 * B, NUM_MATCHES)

    out = pl.pallas_call(
        _stats_body,
        out_shape=[jax.ShapeDtypeStruct((1, 1), jnp.float32)] * 4,
        interpret=interpret,
    )(fi_all, fc_all, x1_glob, x2_glob)
    loss, gl, loc, feat = (o[0, 0] for o in out)
    return loss, gl, loc, feat


def kernel(x1_maps, x2_maps, x1_glob, x2_glob, x1_locations, x2_locations):
    return _run(x1_maps, x2_maps, x1_glob, x2_glob,
                x1_locations, x2_locations)


# speedup vs baseline: 1.2469x; 1.1895x over previous
import functools

import jax
import jax.numpy as jnp
from jax.experimental import pallas as pl

NUM_MATCHES = 20
ALPHA = 0.75
INV_COEFF = 25.0
VAR_COEFF = 15.0
COV_COEFF = 1.0
GAMMA = 1.0

B, P, D_LOC, D_GLOB = 32, 576, 384, 2048
BIG_I32 = 1 << 30


def _dist_matrix(a, b):
    a2 = jnp.sum(a * a, axis=1, keepdims=True)
    b2 = jnp.transpose(jnp.sum(b * b, axis=1, keepdims=True))
    nab2 = jax.lax.dot_general(
        a.astype(jnp.bfloat16), b.astype(jnp.bfloat16) * jnp.bfloat16(-2.0),
        (((1,), (1,)), ((), ())),
        preferred_element_type=jnp.float32)
    d2 = (a2 + b2) + nab2
    return jnp.sqrt(jnp.maximum(d2, 0.0))


def _min_argmin(d, axis):
    iota = jax.lax.broadcasted_iota(jnp.int32, d.shape, axis)
    m = jnp.min(d, axis=axis, keepdims=True)
    arg = jnp.min(jnp.where(d == m, iota, BIG_I32), axis=axis)
    return jnp.min(d, axis=axis), arg


def _knn_body(x1m_ref, x2m_ref, x1l_ref, x2l_ref,
              frmin_ref, frarg_ref, fcmin_ref, fcarg_ref,
              lrmin_ref, lrarg_ref, lcmin_ref, lcarg_ref):
    fd = _dist_matrix(x1m_ref[0], x2m_ref[0])
    frmin_ref[0, 0, :], frarg_ref[0, 0, :] = _min_argmin(fd, 1)
    fcmin_ref[0, 0, :], fcarg_ref[0, 0, :] = _min_argmin(fd, 0)
    ld = _dist_matrix(x1l_ref[0], x2l_ref[0])
    lrmin_ref[0, 0, :], lrarg_ref[0, 0, :] = _min_argmin(ld, 1)
    lcmin_ref[0, 0, :], lcarg_ref[0, 0, :] = _min_argmin(ld, 0)


def _match_terms4(vals, args, ch_in, ch_cand):
    s = 4 * B
    iota = jax.lax.broadcasted_iota(jnp.int32, (s, P), 1)
    v = vals
    fis = []
    fcs = []
    for _ in range(NUM_MATCHES):
        m = jnp.min(v, axis=1, keepdims=True)
        first = jnp.min(jnp.where(v == m, iota, BIG_I32),
                        axis=1, keepdims=True)
        onehot = iota == first
        fi = jnp.sum(jnp.where(onehot, ch_in, 0.0), axis=1, keepdims=True)
        cand = jnp.sum(jnp.where(onehot, args, 0), axis=1, keepdims=True)
        fc = jnp.sum(jnp.where(iota == cand, ch_cand, 0.0),
                     axis=1, keepdims=True)
        fis.append(fi)
        fcs.append(fc)
        v = jnp.where(onehot, jnp.inf, v)
    fi_all = jnp.concatenate(fis, axis=1)
    fc_all = jnp.concatenate(fcs, axis=1)
    terms = []
    for d in range(4):
        fi = fi_all[d * B:(d + 1) * B]
        fc = fc_all[d * B:(d + 1) * B]
        inv = INV_COEFF * jnp.sum((fi - fc) ** 2) / (B * NUM_MATCHES)
        mu_i = jnp.sum(fi, axis=0, keepdims=True) / B
        std_i = jnp.sqrt(jnp.sum((fi - mu_i) ** 2, axis=0,
                                 keepdims=True) / (B - 1))
        mu_c = jnp.sum(fc, axis=0, keepdims=True) / B
        std_c = jnp.sqrt(jnp.sum((fc - mu_c) ** 2, axis=0,
                                 keepdims=True) / (B - 1))
        var = VAR_COEFF * (jnp.sum(jnp.maximum(GAMMA - std_i, 0.0))
                           + jnp.sum(jnp.maximum(GAMMA - std_c, 0.0))
                           ) / (2.0 * NUM_MATCHES)
        terms.append(inv + var)
    return terms


def _global_half(x):
    xc = x - jnp.sum(x, axis=0, keepdims=True) / B
    xcc = xc - jnp.sum(xc, axis=0, keepdims=True) / B
    std = jnp.sqrt(jnp.sum(xcc * xcc, axis=0, keepdims=True) / (B - 1))
    var = jnp.sum(jnp.maximum(GAMMA - std, 0.0)) / D_GLOB
    xcb = xc.astype(jnp.bfloat16)
    xcb32 = xcb.astype(jnp.float32)
    colss = jnp.sum(xcb32 * xcb32, axis=0, keepdims=True)
    g = jax.lax.dot_general(
        xcb, xcb, (((1,), (1,)), ((), ())),
        preferred_element_type=jnp.float32)
    s_all = jnp.sum(g * g)
    s_diag = jnp.sum(colss * colss)
    cov = (s_all - s_diag) / ((B - 1.0) * (B - 1.0)) / D_GLOB
    return var, cov


def _loss_body(vals_ref, args_ref, chin_ref, chcand_ref, g1_ref, g2_ref,
               loss_ref, gl_ref, loc_ref, feat_ref):
    f12, f21, l12, l21 = _match_terms4(
        vals_ref[...], args_ref[...], chin_ref[...], chcand_ref[...])
    feat = (f12 + f21) / 2.0
    loc = (l12 + l21) / 2.0

    g1 = g1_ref[...]
    g2 = g2_ref[...]
    inv_g = INV_COEFF * jnp.sum((g1 - g2) ** 2) / (B * D_GLOB)
    var1, cov1 = _global_half(g1)
    var2, cov2 = _global_half(g2)
    global_loss = (inv_g + VAR_COEFF * (var1 + var2) / 2.0
                   + COV_COEFF * (cov1 + cov2) / 2.0)

    loss = ALPHA * global_loss + (1.0 - ALPHA) * (feat + loc) / 2.0
    loss_ref[...] = jnp.reshape(loss, (1, 1))
    gl_ref[...] = jnp.reshape(global_loss, (1, 1))
    loc_ref[...] = jnp.reshape(loc, (1, 1))
    feat_ref[...] = jnp.reshape(feat, (1, 1))


@functools.partial(jax.jit, static_argnames=("interpret",))
def _run(x1_maps, x2_maps, x1_glob, x2_glob, x1_locations, x2_locations,
         interpret=False):
    fspec = pl.BlockSpec((1, 1, P), lambda b: (b, 0, 0))
    knn = pl.pallas_call(
        _knn_body,
        grid=(B,),
        in_specs=[
            pl.BlockSpec((1, P, D_LOC), lambda b: (b, 0, 0)),
            pl.BlockSpec((1, P, D_LOC), lambda b: (b, 0, 0)),
            pl.BlockSpec((1, P, 2), lambda b: (b, 0, 0)),
            pl.BlockSpec((1, P, 2), lambda b: (b, 0, 0)),
        ],
        out_specs=[fspec] * 8,
        out_shape=(
            [jax.ShapeDtypeStruct((B, 1, P), jnp.float32),
             jax.ShapeDtypeStruct((B, 1, P), jnp.int32)] * 4),
        interpret=interpret,
    )
    (frmin, frarg, fcmin, fcarg,
     lrmin, lrarg, lcmin, lcarg) = knn(
        x1_maps, x2_maps, x1_locations, x2_locations)

    vals = jnp.concatenate(
        [a.reshape(B, P) for a in (frmin, fcmin, lrmin, lcmin)], axis=0)
    args = jnp.concatenate(
        [a.reshape(B, P) for a in (frarg, fcarg, lrarg, lcarg)], axis=0)
    ch1 = x1_maps[:, :, 0]
    ch2 = x2_maps[:, :, 0]
    chin = jnp.concatenate([ch1, ch2, ch1, ch2], axis=0)
    chcand = jnp.concatenate([ch2, ch1, ch2, ch1], axis=0)

    out = pl.pallas_call(
        _loss_body,
        out_shape=[jax.ShapeDtypeStruct((1, 1), jnp.float32)] * 4,
        interpret=interpret,
    )(vals, args, chin, chcand, x1_glob, x2_glob)
    loss, gl, loc, feat = (o[0, 0] for o in out)
    return loss, gl, loc, feat


def kernel(x1_maps, x2_maps, x1_glob, x2_glob, x1_locations, x2_locations):
    return _run(x1_maps, x2_maps, x1_glob, x2_glob,
                x1_locations, x2_locations)
